# baseline (speedup 1.0000x reference)
import functools

import jax
import jax.numpy as jnp
from jax import lax
from jax.experimental import pallas as pl
from jax.experimental.pallas import tpu as pltpu
from jax.experimental.pallas import tpu_sc as plsc

VOCAB = 1000000
DIM = 16
BATCH = 16384

NUM_CORES = 2
NUM_SUBCORES = 16
NW = NUM_CORES * NUM_SUBCORES
BPW = BATCH // NW
CHUNK = 128
NCHUNK = BPW // CHUNK


@functools.partial(
    pl.kernel,
    mesh=plsc.VectorSubcoreMesh(core_axis_name="c", subcore_axis_name="s"),
    out_type=[
        jax.ShapeDtypeStruct((BATCH, DIM), jnp.float32),
        jax.ShapeDtypeStruct((BATCH, DIM), jnp.float32),
    ],
    scratch_types=[
        pltpu.VMEM((NCHUNK, CHUNK), jnp.int32),
        pltpu.VMEM((NCHUNK, CHUNK), jnp.int32),
        pltpu.VMEM((BPW, DIM), jnp.float32),
        pltpu.VMEM((BPW, DIM), jnp.float32),
        pltpu.SemaphoreType.DMA,
    ],
    compiler_params=pltpu.CompilerParams(use_tc_tiling_on_sc=False),
)
def _emb_lookup(user_hbm, item_hbm, wu_hbm, wi_hbm, ou_hbm, oi_hbm,
                idx_u, idx_i, rows_u, rows_i, sem):
    wid = lax.axis_index("s") * NUM_CORES + lax.axis_index("c")
    base = wid * BPW

    pltpu.sync_copy(user_hbm.at[wid], idx_u)
    pltpu.sync_copy(item_hbm.at[wid], idx_i)

    copies = []
    for c in range(NCHUNK):
        copies.append(pltpu.async_copy(
            wu_hbm.at[idx_u.at[c]], rows_u.at[pl.ds(c * CHUNK, CHUNK)], sem))
        copies.append(pltpu.async_copy(
            wi_hbm.at[idx_i.at[c]], rows_i.at[pl.ds(c * CHUNK, CHUNK)], sem))
    for cp in copies:
        cp.wait()

    pltpu.sync_copy(rows_u, ou_hbm.at[pl.ds(base, BPW)])
    pltpu.sync_copy(rows_i, oi_hbm.at[pl.ds(base, BPW)])


def kernel(user, item, W_user, W_item):
    u = user.astype(jnp.int32).reshape(NW, NCHUNK, CHUNK)
    it = item.astype(jnp.int32).reshape(NW, NCHUNK, CHUNK)
    out_user, out_item = _emb_lookup(u, it, W_user, W_item)
    return out_user, out_item


# speedup vs baseline: 7.3476x; 7.3476x over previous
import functools

import jax
import jax.numpy as jnp
from jax import lax
from jax.experimental import pallas as pl
from jax.experimental.pallas import tpu as pltpu
from jax.experimental.pallas import tpu_sc as plsc

VOCAB = 1000000
DIM = 16
BATCH = 16384
LANE = 128

NUM_CORES = 2
NUM_SUBCORES = 16
NW = NUM_CORES * NUM_SUBCORES
BPW = BATCH // NW
G = 16
NG = BPW // G
NGT = 2 * NG
L = 16


def _fire_group(wt_hbm, idxv, slabs, sem, gg, slot):
    kv = idxv[pl.ds(gg * G, G)]
    for b in range(G):
        k = kv[b]
        off = pl.multiple_of(lax.shift_right_logical(k, 7) * LANE, LANE)
        pltpu.async_copy(wt_hbm.at[:, pl.ds(off, LANE)],
                         slabs.at[slot, b], sem)


def _drain_group(wt_hbm, slabs, sem, slot):
    for b in range(G):
        pltpu.make_async_copy(wt_hbm.at[:, pl.ds(0, LANE)],
                              slabs.at[slot, b], sem).wait()


def _extract_group(idxv, slabs, blk, gg, slot):
    kv = idxv[pl.ds(gg * G, G)]
    cv = lax.bitwise_and(kv, LANE - 1)
    r_vec = lax.iota(jnp.int32, L)
    slot_vec = jnp.full((L,), slot, jnp.int32)
    for b in range(G):
        val = plsc.load_gather(
            slabs, [slot_vec, jnp.full((L,), b, jnp.int32), r_vec,
                    jnp.full((L,), cv[b], jnp.int32)])
        plsc.store_scatter(blk, [r_vec, jnp.full((L,), gg * G + b, jnp.int32)],
                           val)


@functools.partial(
    pl.kernel,
    mesh=plsc.VectorSubcoreMesh(core_axis_name="c", subcore_axis_name="s"),
    out_type=[
        jax.ShapeDtypeStruct((DIM, BATCH), jnp.float32),
        jax.ShapeDtypeStruct((DIM, BATCH), jnp.float32),
    ],
    scratch_types=[
        pltpu.VMEM((BPW,), jnp.int32),
        pltpu.VMEM((BPW,), jnp.int32),
        pltpu.VMEM((2, G, DIM, LANE), jnp.float32),
        pltpu.VMEM((DIM, BPW), jnp.float32),
        pltpu.VMEM((DIM, BPW), jnp.float32),
        pltpu.SemaphoreType.DMA,
    ],
    compiler_params=pltpu.CompilerParams(needs_layout_passes=False),
)
def _emb_lookup(user_hbm, item_hbm, wtu_hbm, wti_hbm, otu_hbm, oti_hbm,
                idxv_u, idxv_i, slabs, blk_u, blk_i, sem):
    wid = lax.axis_index("s") * NUM_CORES + lax.axis_index("c")
    base = wid * BPW

    pltpu.sync_copy(user_hbm.at[wid], idxv_u)
    pltpu.sync_copy(item_hbm.at[wid], idxv_i)

    _fire_group(wtu_hbm, idxv_u, slabs, sem, 0, 0)

    def body(j, _):
        slot = lax.rem(j, 2)
        nslot = 1 - slot
        gg = lax.div(j, 2)
        ngg = lax.div(j + 1, 2)

        @pl.when(jnp.logical_and(j + 1 < NGT, lax.rem(j + 1, 2) == 1))
        def _():
            _fire_group(wti_hbm, idxv_i, slabs, sem, ngg, nslot)

        @pl.when(jnp.logical_and(j + 1 < NGT, lax.rem(j + 1, 2) == 0))
        def _():
            _fire_group(wtu_hbm, idxv_u, slabs, sem, ngg, nslot)

        _drain_group(wtu_hbm, slabs, sem, slot)

        @pl.when(lax.rem(j, 2) == 0)
        def _():
            _extract_group(idxv_u, slabs, blk_u, gg, slot)

        @pl.when(lax.rem(j, 2) == 1)
        def _():
            _extract_group(idxv_i, slabs, blk_i, gg, slot)

        return _

    lax.fori_loop(0, NGT, body, None)

    pltpu.sync_copy(blk_u, otu_hbm.at[:, pl.ds(base, BPW)])
    pltpu.sync_copy(blk_i, oti_hbm.at[:, pl.ds(base, BPW)])


def kernel(user, item, W_user, W_item):
    u = user.astype(jnp.int32).reshape(NW, BPW)
    it = item.astype(jnp.int32).reshape(NW, BPW)
    out_u_t, out_i_t = _emb_lookup(u, it, W_user.T, W_item.T)
    return out_u_t.T, out_i_t.T
